# Initial kernel scaffold; baseline (speedup 1.0000x reference)
import jax
import jax.numpy as jnp
from jax.experimental import pallas as pl


def kernel(x, adj1_indices, adj1_values, adj2_indices, adj2_values, W_embed, b_embed, W_cls, b_cls):
    raise NotImplementedError("write your pallas kernel here")


# speedup vs baseline: 1.7671x; 1.7671x over previous
import functools

import jax
import jax.numpy as jnp
from jax import lax
from jax.experimental import pallas as pl
from jax.experimental.pallas import tpu as pltpu
from jax.experimental.pallas import tpu_sc as plsc

N = 10000
NP = 10240
D = 128
H = 128
OUT = 64
HW = 64
NC = 2
NS = 16
CHUNK = 128
RPT = NP // NS
ROWB = 1000

E1 = 320000
E2 = 640000
CH1 = -(-E1 // (NS * CHUNK))
CH2 = -(-E2 // (NS * CHUNK))


def _pad_reshape(a, ch, fill):
    pad = ch * NS * CHUNK - a.shape[0]
    a = jnp.concatenate([a, jnp.full((pad,), fill, a.dtype)])
    return a.reshape(ch, NS, CHUNK)


def _prep_edges(indices, values, ch):
    rows = _pad_reshape(indices[0].astype(jnp.int32), ch, 0)
    cols = _pad_reshape(indices[1].astype(jnp.int32), ch, 0)
    vals = _pad_reshape(values.astype(jnp.float32), ch, 0.0)
    return rows, cols, vals


def _embed_body(x_ref, w_ref, b_ref, o_ref):
    acc = jnp.dot(x_ref[...], w_ref[0], preferred_element_type=jnp.float32)
    o_ref[0] = jnp.maximum(acc + b_ref[0], 0.0)


def _embed(x, w, b):
    ws = jnp.stack([w[:, :HW], w[:, HW:]])
    bs = jnp.stack([b[:HW], b[HW:]]).reshape(NC, 1, HW)
    out = pl.pallas_call(
        _embed_body,
        grid=(NC, N // ROWB),
        in_specs=[
            pl.BlockSpec((ROWB, D), lambda c, i: (i, 0)),
            pl.BlockSpec((1, D, HW), lambda c, i: (c, 0, 0)),
            pl.BlockSpec((1, 1, HW), lambda c, i: (c, 0, 0)),
        ],
        out_specs=pl.BlockSpec((1, ROWB, HW), lambda c, i: (c, i, 0)),
        out_shape=jax.ShapeDtypeStruct((NC, NP, HW), jnp.float32),
    )(x, ws, bs)
    return out.reshape(NC * NP, HW)


def _sc_body(table, r1, c1, v1, r2, c2, v2, out1, out2,
             acc1, acc2, zbuf, colsv, rowsv, valsv, gathv, gsem):
    c = lax.axis_index("c")
    s = lax.axis_index("s")
    base = s * RPT
    zv = jnp.zeros((16,), jnp.float32)

    @pl.loop(0, CHUNK)
    def _z(i):
        for q in range(HW // 16):
            zbuf[i, pl.ds(q * 16, 16)] = zv

    nfull = RPT // CHUNK
    for acc in (acc1, acc2):
        for k in range(nfull):
            pltpu.sync_copy(zbuf.at[...], acc.at[pl.ds(base + k * CHUNK, CHUNK)])
    plsc.subcore_barrier()

    coff = jnp.full((16,), c * NP, jnp.int32)

    def run_adj(rr, cc, vv, nch, acc):
        @pl.loop(0, nch)
        def _(i):
            pltpu.sync_copy(cc.at[i, s], colsv.at[0])
            pltpu.sync_copy(rr.at[i, s], rowsv.at[0])
            pltpu.sync_copy(vv.at[i, s], valsv.at[0])
            for q in range(CHUNK // 16):
                sl = pl.ds(q * 16, 16)
                colsv[0, sl] = colsv[0, sl] + coff
            pltpu.async_copy(table.at[colsv.at[0]], gathv.at[0], gsem).wait()

            @pl.loop(0, CHUNK // 16)
            def _g(g):
                vgrp = valsv[0, pl.ds(g * 16, 16)]
                for e in range(16):
                    vvec = jnp.take_along_axis(
                        vgrp, jnp.full((16,), e, jnp.int32), axis=0)
                    row = g * 16 + e
                    for q in range(HW // 16):
                        sl = pl.ds(q * 16, 16)
                        gathv[0, row, sl] = gathv[0, row, sl] * vvec

            pltpu.sync_copy(gathv.at[0], acc.at[rowsv.at[0]], add=True)

    run_adj(r1, c1, v1, CH1, acc1)
    run_adj(r2, c2, v2, CH2, acc2)
    plsc.subcore_barrier()

    for acc, out in ((acc1, out1), (acc2, out2)):
        for k in range(nfull):
            pltpu.sync_copy(acc.at[pl.ds(base + k * CHUNK, CHUNK)],
                            gathv.at[0])

            @pl.loop(0, CHUNK)
            def _r(i):
                for q in range(HW // 16):
                    sl = pl.ds(q * 16, 16)
                    gathv[0, i, sl] = jnp.maximum(gathv[0, i, sl], 0.0)

            pltpu.sync_copy(gathv.at[0],
                            out.at[pl.ds(c * NP + base + k * CHUNK, CHUNK)])


_sc_layer_call = pl.kernel(
    _sc_body,
    out_type=[jax.ShapeDtypeStruct((NC * NP, HW), jnp.float32)] * 2,
    mesh=plsc.VectorSubcoreMesh(core_axis_name="c", subcore_axis_name="s",
                                num_cores=NC, num_subcores=NS),
    scratch_types=[
        pltpu.VMEM_SHARED((NP, HW), jnp.float32),
        pltpu.VMEM_SHARED((NP, HW), jnp.float32),
        pltpu.VMEM((CHUNK, HW), jnp.float32),
        pltpu.VMEM((1, CHUNK), jnp.int32),
        pltpu.VMEM((1, CHUNK), jnp.int32),
        pltpu.VMEM((1, CHUNK), jnp.float32),
        pltpu.VMEM((1, CHUNK, HW), jnp.float32),
        pltpu.SemaphoreType.DMA,
    ],
    compiler_params=pltpu.CompilerParams(use_tc_tiling_on_sc=False),
)


def _sc_layer(table, e1, e2):
    r1, c1, v1 = e1
    r2, c2, v2 = e2
    return _sc_layer_call(table, r1, c1, v1, r2, c2, v2)


def _cls_body(*refs):
    ins = refs[:14]
    w_ref, b_ref, o_ref = refs[14], refs[15], refs[16]
    hcat = jnp.concatenate([r[0] for r in ins], axis=1)
    acc = jnp.dot(hcat, w_ref[...], preferred_element_type=jnp.float32)
    o_ref[...] = acc + b_ref[...][None, :]


def _classify(pieces, w_cls, b_cls):
    nb = N // ROWB
    in_specs = []
    args = []
    for p in pieces:
        p3 = p.reshape(NC, NP, HW)
        for c in range(NC):
            in_specs.append(
                pl.BlockSpec((1, ROWB, HW), functools.partial(
                    lambda i, cc: (cc, i, 0), cc=c)))
            args.append(p3)
    in_specs.append(pl.BlockSpec(((2 ** 3 - 1) * H, OUT), lambda i: (0, 0)))
    in_specs.append(pl.BlockSpec((OUT,), lambda i: (0,)))
    return pl.pallas_call(
        _cls_body,
        grid=(nb,),
        in_specs=in_specs,
        out_specs=pl.BlockSpec((ROWB, OUT), lambda i: (i, 0)),
        out_shape=jax.ShapeDtypeStruct((N, OUT), jnp.float32),
    )(*args, w_cls, b_cls)


def kernel(x, adj1_indices, adj1_values, adj2_indices, adj2_values,
           W_embed, b_embed, W_cls, b_cls):
    e1 = _prep_edges(adj1_indices, adj1_values, CH1)
    e2 = _prep_edges(adj2_indices, adj2_values, CH2)
    h0 = _embed(x, W_embed, b_embed)
    r1a, r1b = _sc_layer(h0, e1, e2)
    z1a, z2a = _sc_layer(r1a, e1, e2)
    z1b, z2b = _sc_layer(r1b, e1, e2)
    pieces = [h0, r1a, r1b, z1a, z1b, z2a, z2b]
    return _classify(pieces, W_cls, b_cls)


# speedup vs baseline: 5.2883x; 2.9926x over previous
import functools

import jax
import jax.numpy as jnp
from jax import lax
from jax.experimental import pallas as pl
from jax.experimental.pallas import tpu as pltpu
from jax.experimental.pallas import tpu_sc as plsc

N = 10000
NP = 10240
D = 128
H = 128
OUT = 64
HW = 64
NC = 2
NS = 16
CHUNK = 128
RPT = NP // NS
ROWB = 1000

E1 = 320000
E2 = 640000
NBUF = 4


def _nchunks(e):
    ch = -(-e // (NS * CHUNK))
    return -(-ch // NBUF) * NBUF


CH1 = _nchunks(E1)
CH2 = _nchunks(E2)


def _pad_reshape(a, ch, fill):
    pad = ch * NS * CHUNK - a.shape[0]
    a = jnp.concatenate([a, jnp.full((pad,), fill, a.dtype)])
    return a.reshape(ch, NS, CHUNK)


def _prep_edges(indices, values, ch):
    rows = _pad_reshape(indices[0].astype(jnp.int32), ch, 0)
    cols = _pad_reshape(indices[1].astype(jnp.int32), ch, 0)
    vals = _pad_reshape(
        lax.bitcast_convert_type(values.astype(jnp.float32), jnp.int32), ch, 0)
    return jnp.stack([cols, rows, vals], axis=2)


def _embed_body(x_ref, w_ref, b_ref, o_ref):
    acc = jnp.dot(x_ref[...], w_ref[0], preferred_element_type=jnp.float32)
    o_ref[0] = jnp.maximum(acc + b_ref[0], 0.0)


def _embed(x, w, b):
    ws = jnp.stack([w[:, :HW], w[:, HW:]])
    bs = jnp.stack([b[:HW], b[HW:]]).reshape(NC, 1, HW)
    out = pl.pallas_call(
        _embed_body,
        grid=(NC, N // ROWB),
        in_specs=[
            pl.BlockSpec((ROWB, D), lambda c, i: (i, 0)),
            pl.BlockSpec((1, D, HW), lambda c, i: (c, 0, 0)),
            pl.BlockSpec((1, 1, HW), lambda c, i: (c, 0, 0)),
        ],
        out_specs=pl.BlockSpec((1, ROWB, HW), lambda c, i: (c, i, 0)),
        out_shape=jax.ShapeDtypeStruct((NC, NP, HW), jnp.float32),
    )(x, ws, bs)
    return out.reshape(NC * NP, HW)


def _sc_body(table, e1, e2, out1, out2,
             acc1, acc2, edgv, gathv,
             isems, gsems, ssems):
    c = lax.axis_index("c")
    s = lax.axis_index("s")
    base = s * RPT
    zv = jnp.zeros((16,), jnp.float32)

    @pl.loop(0, CHUNK)
    def _z(i):
        for q in range(HW // 16):
            gathv[0, i, pl.ds(q * 16, 16)] = zv

    nfull = RPT // CHUNK
    for acc in (acc1, acc2):
        for k in range(nfull):
            pltpu.sync_copy(gathv.at[0],
                            acc.at[pl.ds(base + k * CHUNK, CHUNK)])
    plsc.subcore_barrier()

    coff = jnp.full((16,), c * NP, jnp.int32)

    def run_adj(ee, nch, acc):
        def idx_post(i, b):
            pltpu.async_copy(ee.at[i, s], edgv.at[b], isems.at[b])

        def idx_wait(i, b):
            pltpu.make_async_copy(ee.at[i, s], edgv.at[b], isems.at[b]).wait()

        def gather_post(b):
            pltpu.async_copy(table.at[edgv.at[b, 0]], gathv.at[b],
                             gsems.at[b])

        def gather_wait(b):
            pltpu.make_async_copy(
                table.at[edgv.at[b, 0]], gathv.at[b], gsems.at[b]).wait()

        def scatter_post(b):
            pltpu.async_copy(gathv.at[b], acc.at[edgv.at[b, 1]], ssems.at[b],
                             add=True)

        def scatter_wait(b):
            pltpu.make_async_copy(
                gathv.at[b], acc.at[edgv.at[b, 1]], ssems.at[b]).wait()

        def offset(b):
            for q in range(CHUNK // 16):
                sl = pl.ds(q * 16, 16)
                edgv[b, 0, sl] = edgv[b, 0, sl] + coff

        def scale(b):
            @plsc.parallel_loop(0, CHUNK // 16, unroll=2)
            def _g(g):
                vgrp = plsc.bitcast(
                    edgv[b, 2, pl.ds(g * 16, 16)], jnp.float32)
                for e in range(16):
                    vvec = jnp.take_along_axis(
                        vgrp, jnp.full((16,), e, jnp.int32), axis=0)
                    row = g * 16 + e
                    xs = [gathv[b, row, pl.ds(q * 16, 16)]
                          for q in range(HW // 16)]
                    for q in range(HW // 16):
                        gathv[b, row, pl.ds(q * 16, 16)] = xs[q] * vvec

        idx_post(0, 0)
        idx_post(1, 1)
        idx_wait(0, 0)
        offset(0)
        gather_post(0)

        @pl.loop(0, nch // NBUF)
        def _(g):
            for b in range(NBUF):
                i = g * NBUF + b

                @pl.when(i >= 2)
                def _w():
                    scatter_wait((b + 2) % NBUF)

                @pl.when(i + 2 < nch)
                def _p():
                    idx_post(i + 2, (b + 2) % NBUF)

                @pl.when(i + 1 < nch)
                def _n():
                    idx_wait(i + 1, (b + 1) % NBUF)
                    offset((b + 1) % NBUF)
                    gather_post((b + 1) % NBUF)

                gather_wait(b)
                scale(b)
                scatter_post(b)

        scatter_wait((nch - 2) % NBUF)
        scatter_wait((nch - 1) % NBUF)

    run_adj(e1, CH1, acc1)
    run_adj(e2, CH2, acc2)
    plsc.subcore_barrier()

    for acc, out in ((acc1, out1), (acc2, out2)):
        for k in range(nfull):
            pltpu.sync_copy(acc.at[pl.ds(base + k * CHUNK, CHUNK)],
                            gathv.at[0])

            @plsc.parallel_loop(0, CHUNK, unroll=2)
            def _r(i):
                xs = [gathv[0, i, pl.ds(q * 16, 16)]
                      for q in range(HW // 16)]
                for q in range(HW // 16):
                    gathv[1, i, pl.ds(q * 16, 16)] = jnp.maximum(xs[q], 0.0)

            pltpu.sync_copy(gathv.at[1],
                            out.at[pl.ds(c * NP + base + k * CHUNK, CHUNK)])


_sc_layer_call = pl.kernel(
    _sc_body,
    out_type=[jax.ShapeDtypeStruct((NC * NP, HW), jnp.float32)] * 2,
    mesh=plsc.VectorSubcoreMesh(core_axis_name="c", subcore_axis_name="s",
                                num_cores=NC, num_subcores=NS),
    scratch_types=[
        pltpu.VMEM_SHARED((NP, HW), jnp.float32),
        pltpu.VMEM_SHARED((NP, HW), jnp.float32),
        pltpu.VMEM((NBUF, 3, CHUNK), jnp.int32),
        pltpu.VMEM((NBUF, CHUNK, HW), jnp.float32),
        pltpu.SemaphoreType.DMA((NBUF,)),
        pltpu.SemaphoreType.DMA((NBUF,)),
        pltpu.SemaphoreType.DMA((NBUF,)),
    ],
    compiler_params=pltpu.CompilerParams(use_tc_tiling_on_sc=False,
                                         needs_layout_passes=False),
)


def _sc_layer(table, e1, e2):
    return _sc_layer_call(table, e1, e2)


def _cls_body(*refs):
    ins = refs[:14]
    w_ref, b_ref, o_ref = refs[14], refs[15], refs[16]
    hcat = jnp.concatenate([r[0] for r in ins], axis=1)
    acc = jnp.dot(hcat, w_ref[...], preferred_element_type=jnp.float32)
    o_ref[...] = acc + b_ref[...][None, :]


def _classify(pieces, w_cls, b_cls):
    nb = N // ROWB
    in_specs = []
    args = []
    for p in pieces:
        p3 = p.reshape(NC, NP, HW)
        for c in range(NC):
            in_specs.append(
                pl.BlockSpec((1, ROWB, HW), functools.partial(
                    lambda i, cc: (cc, i, 0), cc=c)))
            args.append(p3)
    in_specs.append(pl.BlockSpec(((2 ** 3 - 1) * H, OUT), lambda i: (0, 0)))
    in_specs.append(pl.BlockSpec((OUT,), lambda i: (0,)))
    return pl.pallas_call(
        _cls_body,
        grid=(nb,),
        in_specs=in_specs,
        out_specs=pl.BlockSpec((ROWB, OUT), lambda i: (i, 0)),
        out_shape=jax.ShapeDtypeStruct((N, OUT), jnp.float32),
    )(*args, w_cls, b_cls)


def kernel(x, adj1_indices, adj1_values, adj2_indices, adj2_values,
           W_embed, b_embed, W_cls, b_cls):
    e1 = _prep_edges(adj1_indices, adj1_values, CH1)
    e2 = _prep_edges(adj2_indices, adj2_values, CH2)
    h0 = _embed(x, W_embed, b_embed)
    r1a, r1b = _sc_layer(h0, e1, e2)
    z1a, z2a = _sc_layer(r1a, e1, e2)
    z1b, z2b = _sc_layer(r1b, e1, e2)
    pieces = [h0, r1a, r1b, z1a, z1b, z2a, z2b]
    return _classify(pieces, W_cls, b_cls)


# speedup vs baseline: 6.0226x; 1.1389x over previous
import functools

import jax
import jax.numpy as jnp
from jax import lax
from jax.experimental import pallas as pl
from jax.experimental.pallas import tpu as pltpu
from jax.experimental.pallas import tpu_sc as plsc

N = 10000
NP = 10240
D = 128
H = 128
OUT = 64
HW = 64
NC = 2
NS = 16
CHUNK = 128
RPT = NP // NS
ROWB = 1000

E1 = 320000
E2 = 640000
NBUF = 4


def _nchunks(e):
    ch = -(-e // (NS * CHUNK))
    return -(-ch // NBUF) * NBUF


CH1 = _nchunks(E1)
CH2 = _nchunks(E2)


def _pad_reshape(a, ch, fill):
    pad = ch * NS * CHUNK - a.shape[0]
    a = jnp.concatenate([a, jnp.full((pad,), fill, a.dtype)])
    return a.reshape(ch, NS, CHUNK)


def _prep_edges(indices, values, ch):
    rows = _pad_reshape(indices[0].astype(jnp.int32), ch, 0)
    cols = _pad_reshape(indices[1].astype(jnp.int32), ch, 0)
    vals = _pad_reshape(values.astype(jnp.float32), ch, 0.0)
    return rows, cols, vals


def _embed_body(x_ref, w_ref, b_ref, o_ref):
    acc = jnp.dot(x_ref[...], w_ref[0], preferred_element_type=jnp.float32)
    o_ref[0] = jnp.maximum(acc + b_ref[0], 0.0)


def _embed(x, w, b):
    ws = jnp.stack([w[:, :HW], w[:, HW:]])
    bs = jnp.stack([b[:HW], b[HW:]]).reshape(NC, 1, HW)
    out = pl.pallas_call(
        _embed_body,
        grid=(NC, N // ROWB),
        in_specs=[
            pl.BlockSpec((ROWB, D), lambda c, i: (i, 0)),
            pl.BlockSpec((1, D, HW), lambda c, i: (c, 0, 0)),
            pl.BlockSpec((1, 1, HW), lambda c, i: (c, 0, 0)),
        ],
        out_specs=pl.BlockSpec((1, ROWB, HW), lambda c, i: (c, i, 0)),
        out_shape=jax.ShapeDtypeStruct((NC, NP, HW), jnp.float32),
    )(x, ws, bs)
    return out.reshape(NC * NP, HW)


def _sc_body(table, r1, c1, v1, r2, c2, v2, out1, out2,
             acc1, acc2, colsv, rowsv, valsv, gathv,
             isems, gsems, ssems):
    c = lax.axis_index("c")
    s = lax.axis_index("s")
    base = s * RPT
    zv = jnp.zeros((16,), jnp.float32)

    @pl.loop(0, CHUNK)
    def _z(i):
        for q in range(HW // 16):
            gathv[0, i, pl.ds(q * 16, 16)] = zv

    nfull = RPT // CHUNK
    for acc in (acc1, acc2):
        for k in range(nfull):
            pltpu.sync_copy(gathv.at[0],
                            acc.at[pl.ds(base + k * CHUNK, CHUNK)])
    plsc.subcore_barrier()

    coff = jnp.full((16,), c * NP, jnp.int32)

    def run_adj(rr, cc, vv, nch, acc):
        def idx_post(i, b):
            pltpu.async_copy(cc.at[i, s], colsv.at[b], isems.at[b])
            pltpu.async_copy(rr.at[i, s], rowsv.at[b], isems.at[b])
            pltpu.async_copy(vv.at[i, s], valsv.at[b], isems.at[b])

        def idx_wait(i, b):
            pltpu.make_async_copy(cc.at[i, s], colsv.at[b], isems.at[b]).wait()
            pltpu.make_async_copy(rr.at[i, s], rowsv.at[b], isems.at[b]).wait()
            pltpu.make_async_copy(vv.at[i, s], valsv.at[b], isems.at[b]).wait()

        def gather_post(b):
            pltpu.async_copy(table.at[colsv.at[b]], gathv.at[b], gsems.at[b])

        def gather_wait(b):
            pltpu.make_async_copy(
                table.at[colsv.at[b]], gathv.at[b], gsems.at[b]).wait()

        def scatter_post(b):
            pltpu.async_copy(gathv.at[b], acc.at[rowsv.at[b]], ssems.at[b],
                             add=True)

        def scatter_wait(b):
            pltpu.make_async_copy(
                gathv.at[b], acc.at[rowsv.at[b]], ssems.at[b]).wait()

        def offset(b):
            for q in range(CHUNK // 16):
                sl = pl.ds(q * 16, 16)
                colsv[b, sl] = colsv[b, sl] + coff

        def scale(b):
            @plsc.parallel_loop(0, CHUNK // 16, unroll=2)
            def _g(g):
                vgrp = valsv[b, pl.ds(g * 16, 16)]
                for e in range(16):
                    vvec = jnp.take_along_axis(
                        vgrp, jnp.full((16,), e, jnp.int32), axis=0)
                    row = g * 16 + e
                    xs = [gathv[b, row, pl.ds(q * 16, 16)]
                          for q in range(HW // 16)]
                    for q in range(HW // 16):
                        gathv[b, row, pl.ds(q * 16, 16)] = xs[q] * vvec

        idx_post(0, 0)
        idx_post(1, 1)
        idx_wait(0, 0)
        offset(0)
        gather_post(0)

        @pl.loop(0, nch // NBUF)
        def _(g):
            for b in range(NBUF):
                i = g * NBUF + b

                @pl.when(i >= 2)
                def _w():
                    scatter_wait((b + 2) % NBUF)

                @pl.when(i + 2 < nch)
                def _p():
                    idx_post(i + 2, (b + 2) % NBUF)

                @pl.when(i + 1 < nch)
                def _n():
                    idx_wait(i + 1, (b + 1) % NBUF)
                    offset((b + 1) % NBUF)
                    gather_post((b + 1) % NBUF)

                gather_wait(b)
                scale(b)
                scatter_post(b)

        scatter_wait((nch - 2) % NBUF)
        scatter_wait((nch - 1) % NBUF)

    run_adj(r1, c1, v1, CH1, acc1)
    run_adj(r2, c2, v2, CH2, acc2)
    plsc.subcore_barrier()

    for acc, out in ((acc1, out1), (acc2, out2)):
        for k in range(nfull):
            pltpu.sync_copy(acc.at[pl.ds(base + k * CHUNK, CHUNK)],
                            gathv.at[0])

            @plsc.parallel_loop(0, CHUNK, unroll=2)
            def _r(i):
                xs = [gathv[0, i, pl.ds(q * 16, 16)]
                      for q in range(HW // 16)]
                for q in range(HW // 16):
                    gathv[1, i, pl.ds(q * 16, 16)] = jnp.maximum(xs[q], 0.0)

            pltpu.sync_copy(gathv.at[1],
                            out.at[pl.ds(c * NP + base + k * CHUNK, CHUNK)])


_sc_layer_call = pl.kernel(
    _sc_body,
    out_type=[jax.ShapeDtypeStruct((NC * NP, HW), jnp.float32)] * 2,
    mesh=plsc.VectorSubcoreMesh(core_axis_name="c", subcore_axis_name="s",
                                num_cores=NC, num_subcores=NS),
    scratch_types=[
        pltpu.VMEM_SHARED((NP, HW), jnp.float32),
        pltpu.VMEM_SHARED((NP, HW), jnp.float32),
        pltpu.VMEM((NBUF, CHUNK), jnp.int32),
        pltpu.VMEM((NBUF, CHUNK), jnp.int32),
        pltpu.VMEM((NBUF, CHUNK), jnp.float32),
        pltpu.VMEM((NBUF, CHUNK, HW), jnp.float32),
        pltpu.SemaphoreType.DMA((NBUF,)),
        pltpu.SemaphoreType.DMA((NBUF,)),
        pltpu.SemaphoreType.DMA((NBUF,)),
    ],
    compiler_params=pltpu.CompilerParams(use_tc_tiling_on_sc=False),
)


def _sc_layer(table, e1, e2):
    r1, c1, v1 = e1
    r2, c2, v2 = e2
    return _sc_layer_call(table, r1, c1, v1, r2, c2, v2)


def _cls_body(*refs):
    ins = refs[:14]
    w_ref, b_ref, o_ref = refs[14], refs[15], refs[16]
    hcat = jnp.concatenate([r[0] for r in ins], axis=1)
    acc = jnp.dot(hcat, w_ref[...], preferred_element_type=jnp.float32)
    o_ref[...] = acc + b_ref[...][None, :]


def _classify(pieces, w_cls, b_cls):
    nb = N // ROWB
    in_specs = []
    args = []
    for p in pieces:
        p3 = p.reshape(NC, NP, HW)
        for c in range(NC):
            in_specs.append(
                pl.BlockSpec((1, ROWB, HW), functools.partial(
                    lambda i, cc: (cc, i, 0), cc=c)))
            args.append(p3)
    in_specs.append(pl.BlockSpec(((2 ** 3 - 1) * H, OUT), lambda i: (0, 0)))
    in_specs.append(pl.BlockSpec((OUT,), lambda i: (0,)))
    return pl.pallas_call(
        _cls_body,
        grid=(nb,),
        in_specs=in_specs,
        out_specs=pl.BlockSpec((ROWB, OUT), lambda i: (i, 0)),
        out_shape=jax.ShapeDtypeStruct((N, OUT), jnp.float32),
    )(*args, w_cls, b_cls)


def kernel(x, adj1_indices, adj1_values, adj2_indices, adj2_values,
           W_embed, b_embed, W_cls, b_cls):
    e1 = _prep_edges(adj1_indices, adj1_values, CH1)
    e2 = _prep_edges(adj2_indices, adj2_values, CH2)
    h0 = _embed(x, W_embed, b_embed)
    r1a, r1b = _sc_layer(h0, e1, e2)
    z1a, z2a = _sc_layer(r1a, e1, e2)
    z1b, z2b = _sc_layer(r1b, e1, e2)
    pieces = [h0, r1a, r1b, z1a, z1b, z2a, z2b]
    return _classify(pieces, W_cls, b_cls)
